# baseline (speedup 1.0000x reference)
import functools

import jax
import jax.numpy as jnp
from jax import lax
from jax.experimental import pallas as pl
from jax.experimental.pallas import tpu as pltpu
from jax.experimental.pallas import tpu_sc as plsc

_B, _S, _D = 4, 4096, 2048
_NC, _NS = 2, 16
_NW = _NC * _NS
_ROWS = _B * _S
_RPW = _ROWS // _NW
_CHUNK = 8
_SLOTS = 6
_NCHUNK = _RPW // _CHUNK
_FULL_IT = _NCHUNK // _SLOTS
_TAIL = _NCHUNK - _FULL_IT * _SLOTS
_LANES = 16


def _gather_body(x_hbm, perm_hbm, out_hbm, idx_v, buf_v, *sems):
    sem_g = sems[:_SLOTS]
    sem_s = sems[_SLOTS:]
    cid = lax.axis_index("c")
    sid = lax.axis_index("s")
    wid = sid * _NC + cid
    base = wid * _RPW
    b = base // _S
    i0 = base - b * _S
    off = b * _S

    pltpu.sync_copy(perm_hbm.at[pl.ds(i0, _RPW)], idx_v)
    off_vec = jnp.full((_LANES,), off, dtype=jnp.int32)
    for j in range(_RPW // _LANES):
        sl = pl.ds(j * _LANES, _LANES)
        idx_v[sl] = idx_v[sl] + off_vec

    def g_copy(g, slot):
        idx_slice = idx_v.at[pl.ds(g * _CHUNK, _CHUNK)]
        return pltpu.make_async_copy(
            x_hbm.at[idx_slice], buf_v.at[slot], sem_g[slot])

    def s_copy(g, slot):
        return pltpu.make_async_copy(
            buf_v.at[slot], out_hbm.at[pl.ds(base + g * _CHUNK, _CHUNK)],
            sem_s[slot])

    def phase(g0, nslots):
        for s in range(nslots):
            g_copy(g0 + s, s).start()
        for s in range(nslots):
            g_copy(g0 + s, s).wait()
        for s in range(nslots):
            s_copy(g0 + s, s).start()
        for s in range(nslots):
            s_copy(g0 + s, s).wait()
        plsc.subcore_barrier()

    def it_body(t, carry):
        phase(t * _SLOTS, _SLOTS)
        return carry

    lax.fori_loop(0, _FULL_IT, it_body, 0)
    phase(_FULL_IT * _SLOTS, _TAIL)


@jax.jit
def kernel(x, perm):
    x2 = x.reshape(_ROWS, _D)
    p32 = perm.astype(jnp.int32)
    mesh = plsc.VectorSubcoreMesh(core_axis_name="c", subcore_axis_name="s")
    run = pl.kernel(
        _gather_body,
        mesh=mesh,
        out_type=jax.ShapeDtypeStruct((_ROWS, _D), jnp.float32),
        scratch_types=[
            pltpu.VMEM((_RPW,), jnp.int32),
            pltpu.VMEM((_SLOTS, _CHUNK, _D), jnp.float32),
        ] + [pltpu.SemaphoreType.DMA] * (2 * _SLOTS),
    )
    out = run(x2, p32)
    return out.reshape(_B, _S, _D)


# speedup vs baseline: 1.1078x; 1.1078x over previous
import functools

import jax
import jax.numpy as jnp
from jax import lax
from jax.experimental import pallas as pl
from jax.experimental.pallas import tpu as pltpu
from jax.experimental.pallas import tpu_sc as plsc

_B, _S, _D = 4, 4096, 2048
_NC, _NS = 2, 16
_NW = _NC * _NS
_ROWS = _B * _S
_RPW = _ROWS // _NW
_CHUNK = 8
_NBUF = 6
_LOOK = 3
_NCHUNK = _RPW // _CHUNK
_LANES = 16


def _gather_body(x_hbm, perm_hbm, out_hbm, idx_v, buf_v, *sems):
    sem_g = sems[:_NBUF]
    sem_s = sems[_NBUF:]
    cid = lax.axis_index("c")
    sid = lax.axis_index("s")
    wid = sid * _NC + cid
    base = wid * _RPW
    b = base // _S
    i0 = base - b * _S
    off = b * _S

    pltpu.sync_copy(perm_hbm.at[pl.ds(i0, _RPW)], idx_v)
    off_vec = jnp.full((_LANES,), off, dtype=jnp.int32)
    for j in range(_RPW // _LANES):
        sl = pl.ds(j * _LANES, _LANES)
        idx_v[sl] = idx_v[sl] + off_vec

    def g_copy(g, slot):
        idx_slice = idx_v.at[pl.ds(g * _CHUNK, _CHUNK)]
        return pltpu.make_async_copy(
            x_hbm.at[idx_slice], buf_v.at[slot], sem_g[slot])

    def s_copy(g, slot):
        return pltpu.make_async_copy(
            buf_v.at[slot], out_hbm.at[pl.ds(base + g * _CHUNK, _CHUNK)],
            sem_s[slot])

    def position(g):
        pf = g + _LOOK
        if pf < _NCHUNK:
            if pf - _NBUF >= 0:
                s_copy(pf - _NBUF, pf % _NBUF).wait()
            g_copy(pf, pf % _NBUF).start()
        g_copy(g, g % _NBUF).wait()
        s_copy(g, g % _NBUF).start()

    for g in range(_LOOK):
        g_copy(g, g).start()
    for g in range(7):
        position(g)

    def steady(t, carry):
        for bb in range(_NBUF):
            g = 7 + t * _NBUF + bb
            slot_c = (7 + bb) % _NBUF
            slot_p = (7 + bb + _LOOK) % _NBUF
            s_copy(g + _LOOK - _NBUF, slot_p).wait()
            g_copy(g + _LOOK, slot_p).start()
            g_copy(g, slot_c).wait()
            s_copy(g, slot_c).start()
        return carry

    lax.fori_loop(0, 9, steady, 0)

    for g in range(61, _NCHUNK):
        position(g)
    for g in range(_NCHUNK - _NBUF, _NCHUNK):
        s_copy(g, g % _NBUF).wait()


@jax.jit
def kernel(x, perm):
    x2 = x.reshape(_ROWS, _D)
    p32 = perm.astype(jnp.int32)
    mesh = plsc.VectorSubcoreMesh(core_axis_name="c", subcore_axis_name="s")
    run = pl.kernel(
        _gather_body,
        mesh=mesh,
        out_type=jax.ShapeDtypeStruct((_ROWS, _D), jnp.float32),
        scratch_types=[
            pltpu.VMEM((_RPW,), jnp.int32),
            pltpu.VMEM((_NBUF, _CHUNK, _D), jnp.float32),
        ] + [pltpu.SemaphoreType.DMA] * (2 * _NBUF),
    )
    out = run(x2, p32)
    return out.reshape(_B, _S, _D)


# speedup vs baseline: 1.1399x; 1.0290x over previous
import functools

import jax
import jax.numpy as jnp
from jax import lax
from jax.experimental import pallas as pl
from jax.experimental.pallas import tpu as pltpu
from jax.experimental.pallas import tpu_sc as plsc

_B, _S, _D = 4, 4096, 2048
_NC, _NS = 2, 16
_NW = _NC * _NS
_ROWS = _B * _S
_RPW = _ROWS // _NW
_CHUNK = 8
_NBUF = 4
_NCHUNK = _RPW // _CHUNK
_LANES = 16


def _gather_body(x_hbm, perm_hbm, out_hbm, sp_idx, idx_sm, sp, *sems):
    sem_in = sems[:_NBUF]
    sem_out = sems[_NBUF:]
    cid = lax.axis_index("c")
    sid = lax.axis_index("s")
    wid = sid * _NC + cid
    base = wid * _RPW
    b = base // _S
    i0 = base - b * _S
    off = b * _S

    pltpu.sync_copy(perm_hbm.at[pl.ds(i0, _RPW)], sp_idx.at[sid])
    pltpu.sync_copy(sp_idx.at[sid], idx_sm)

    spbase = sid * (_NBUF * _CHUNK)

    def fill(g, slot):
        for j in range(_CHUNK):
            r = idx_sm[g * _CHUNK + j] + off
            pltpu.make_async_copy(
                x_hbm.at[pl.ds(r, 1)],
                sp.at[pl.ds(spbase + slot * _CHUNK + j, 1)],
                sem_in[slot]).start()

    def drain_fill(g, slot):
        for j in range(_CHUNK):
            pltpu.make_async_copy(
                x_hbm.at[pl.ds(0, 1)],
                sp.at[pl.ds(spbase + slot * _CHUNK + j, 1)],
                sem_in[slot]).wait()

    def out_copy(g, slot):
        return pltpu.make_async_copy(
            sp.at[pl.ds(spbase + slot * _CHUNK, _CHUNK)],
            out_hbm.at[pl.ds(base + g * _CHUNK, _CHUNK)],
            sem_out[slot])

    for s in range(_NBUF):
        fill(s, s)

    def group(i, carry):
        g0 = i * _NBUF
        for s in range(_NBUF):
            drain_fill(g0 + s, s)
            out_copy(g0 + s, s).start()
        for s in range(_NBUF):
            out_copy(g0 + s, s).wait()
            fill(g0 + _NBUF + s, s)
        return carry

    lax.fori_loop(0, _NCHUNK // _NBUF - 1, group, 0)

    gl = (_NCHUNK // _NBUF - 1) * _NBUF
    for s in range(_NBUF):
        drain_fill(gl + s, s)
        out_copy(gl + s, s).start()
    for s in range(_NBUF):
        out_copy(gl + s, s).wait()


@jax.jit
def kernel(x, perm):
    x2 = x.reshape(_ROWS, _D)
    p32 = perm.astype(jnp.int32)
    mesh = plsc.VectorSubcoreMesh(core_axis_name="c", subcore_axis_name="s")
    run = pl.kernel(
        _gather_body,
        mesh=mesh,
        out_type=jax.ShapeDtypeStruct((_ROWS, _D), jnp.float32),
        scratch_types=[
            pltpu.VMEM_SHARED((_NS, _RPW), jnp.int32),
            pltpu.SMEM((_RPW,), jnp.int32),
            pltpu.VMEM_SHARED((_NS * _NBUF * _CHUNK, _D), jnp.float32),
        ] + [pltpu.SemaphoreType.DMA] * (2 * _NBUF),
    )
    out = run(x2, p32)
    return out.reshape(_B, _S, _D)
